# Initial kernel scaffold; baseline (speedup 1.0000x reference)
import jax
import jax.numpy as jnp
from jax.experimental import pallas as pl


def kernel(labels, control_points, stroke_table, startpoint_table, endpoint_table, W, b):
    raise NotImplementedError("write your pallas kernel here")


# speedup vs baseline: 1.3852x; 1.3852x over previous
import functools

import numpy as np
import jax
import jax.numpy as jnp
from jax import lax
from jax.experimental import pallas as pl
from jax.experimental.pallas import tpu as pltpu
from jax.experimental.pallas import tpu_sc as plsc

DIM = 64
DIM_PE = 16
NW = 32
LANE = 128
K = 2
C = K * LANE


def _gather_sum(t0, t1, t2, idx, n_rows):
    cpw = idx.shape[2]
    mesh = plsc.VectorSubcoreMesh(core_axis_name="c", subcore_axis_name="s")

    @functools.partial(
        pl.kernel,
        mesh=mesh,
        compiler_params=pltpu.CompilerParams(use_tc_tiling_on_sc=False),
        out_type=jax.ShapeDtypeStruct((n_rows, DIM), jnp.float32),
        scratch_types=[
            pltpu.VMEM((3, cpw, K, LANE), jnp.int32),
            pltpu.VMEM((3, C, DIM), jnp.float32),
            pltpu.SemaphoreType.DMA,
        ],
    )
    def body(t0_h, t1_h, t2_h, idx_h, out_h, idx_v, g_v, sem):
        wid = lax.axis_index("s") * 2 + lax.axis_index("c")
        for t in range(3):
            pltpu.sync_copy(idx_h.at[t, wid], idx_v.at[t])

        def chunk(ch, carry):
            copies = []
            for t, tbl in enumerate((t0_h, t1_h, t2_h)):
                for j in range(K):
                    copies.append(
                        pltpu.async_copy(
                            tbl.at[idx_v.at[t, ch, j]],
                            g_v.at[t, pl.ds(j * LANE, LANE)],
                            sem,
                        )
                    )
            for cpy in copies:
                cpy.wait()

            def row(r, carry2):
                for seg in range(DIM // 16):
                    s = pl.ds(seg * 16, 16)
                    g_v[0, r, s] = g_v[0, r, s] + g_v[1, r, s] + g_v[2, r, s]
                return carry2

            lax.fori_loop(0, C, row, 0)
            base = (wid * cpw + ch) * C
            pltpu.sync_copy(g_v.at[0], out_h.at[pl.ds(base, C)])
            return carry

        lax.fori_loop(0, cpw, chunk, 0)

    return body(t0, t1, t2, idx)


def _pe_consts():
    m = np.zeros((8, 16 * DIM_PE), np.float32)
    p = np.zeros((1, 16 * DIM_PE), np.float32)
    for j in range(8):
        for t in range(32):
            c = 32 * j + t
            m[j, c] = np.pi * ((t % 16) + 1)
            p[0, c] = 0.0 if t < 16 else np.pi / 2
    return m, p


def _mixer_body(g_ref, cp_ref, l0_ref, m_ref, p_ref, w_ref, b_ref, x_ref, msk_ref):
    phase = jnp.dot(
        cp_ref[...], m_ref[...],
        preferred_element_type=jnp.float32, precision=lax.Precision.HIGHEST,
    )
    pe = jnp.sin(phase + p_ref[...])
    xpe = jnp.dot(
        pe, w_ref[...],
        preferred_element_type=jnp.float32, precision=lax.Precision.HIGHEST,
    ) + b_ref[...]
    x_ref[...] = (g_ref[...] + xpe) * 0.5
    msk_ref[...] = (l0_ref[...] < 0).astype(jnp.int32)


def _mixer(g, cp, l0, w, b, n_rows):
    r_blk = 2048
    m, p = _pe_consts()
    grid = (n_rows // r_blk,)
    return pl.pallas_call(
        _mixer_body,
        grid=grid,
        in_specs=[
            pl.BlockSpec((r_blk, DIM), lambda i: (i, 0)),
            pl.BlockSpec((r_blk, 8), lambda i: (i, 0)),
            pl.BlockSpec((r_blk, 1), lambda i: (i, 0)),
            pl.BlockSpec((8, 256), lambda i: (0, 0)),
            pl.BlockSpec((1, 256), lambda i: (0, 0)),
            pl.BlockSpec((256, DIM), lambda i: (0, 0)),
            pl.BlockSpec((1, DIM), lambda i: (0, 0)),
        ],
        out_specs=[
            pl.BlockSpec((r_blk, DIM), lambda i: (i, 0)),
            pl.BlockSpec((r_blk, 1), lambda i: (i, 0)),
        ],
        out_shape=[
            jax.ShapeDtypeStruct((n_rows, DIM), jnp.float32),
            jax.ShapeDtypeStruct((n_rows, 1), jnp.int32),
        ],
    )(g, cp, l0, jnp.asarray(m), jnp.asarray(p), w, b.reshape(1, DIM))


def kernel(labels, control_points, stroke_table, startpoint_table, endpoint_table, W, b):
    b_, s_ = labels.shape[0], labels.shape[1]
    n = b_ * s_
    cpw = n // (NW * C)
    lab = jnp.maximum(labels.reshape(n, 3), 0)
    idx = lab.T.reshape(3, NW, cpw, K, LANE)
    g = _gather_sum(stroke_table, startpoint_table, endpoint_table, idx, n)
    cp = control_points.reshape(n, 8)
    l0 = labels.reshape(n, 3)[:, 0:1]
    x, msk = _mixer(g, cp, l0, W, b, n)
    return x.reshape(b_, s_, DIM), msk.reshape(b_, s_).astype(jnp.bool_)


# speedup vs baseline: 2.9983x; 2.1645x over previous
import functools

import numpy as np
import jax
import jax.numpy as jnp
from jax import lax
from jax.experimental import pallas as pl
from jax.experimental.pallas import tpu as pltpu
from jax.experimental.pallas import tpu_sc as plsc

DIM = 64
NW = 32
LANE = 128
ROWS_PC = 6
RPC = ROWS_PC * LANE


def _gather3(tcat, idx2d, off, n_tok):
    rows_pw = idx2d.shape[0] // NW
    cpw = rows_pw // ROWS_PC
    mesh = plsc.VectorSubcoreMesh(core_axis_name="c", subcore_axis_name="s")

    @functools.partial(
        pl.kernel,
        mesh=mesh,
        compiler_params=pltpu.CompilerParams(use_tc_tiling_on_sc=False),
        out_type=jax.ShapeDtypeStruct((n_tok * 3, DIM), jnp.float32),
        scratch_types=[
            pltpu.VMEM((rows_pw, LANE), jnp.int32),
            pltpu.VMEM((3, LANE), jnp.int32),
            pltpu.VMEM((RPC, DIM), jnp.float32),
            pltpu.SemaphoreType.DMA,
        ],
    )
    def body(t_h, idx_h, off_h, out_h, idx_v, off_v, g_v, sem):
        wid = lax.axis_index("s") * 2 + lax.axis_index("c")
        pltpu.sync_copy(idx_h.at[pl.ds(wid * rows_pw, rows_pw)], idx_v)
        pltpu.sync_copy(off_h, off_v)

        for r3 in range(3):
            phase = (2 * r3) % 3

            def addbody(i, c, r3=r3, phase=phase):
                row = i * 3 + r3
                for s in range(LANE // 16):
                    sl = pl.ds(s * 16, 16)
                    idx_v[row, sl] = idx_v[row, sl] + off_v[phase, sl]
                return c

            lax.fori_loop(0, rows_pw // 3, addbody, 0)

        def chunk(ch, c):
            cps = []
            for r6 in range(ROWS_PC):
                cps.append(
                    pltpu.async_copy(
                        t_h.at[idx_v.at[ch * ROWS_PC + r6]],
                        g_v.at[pl.ds(r6 * LANE, LANE)],
                        sem,
                    )
                )
            for cp_ in cps:
                cp_.wait()
            pltpu.sync_copy(g_v, out_h.at[pl.ds((wid * cpw + ch) * RPC, RPC)])
            return c

        lax.fori_loop(0, cpw, chunk, 0)

    return body(tcat, idx2d, off)


def _mixer_body(g_ref, cpt_ref, l0_ref, w_ref, b_ref, x_ref, msk_ref):
    t = cpt_ref[...] * np.float32(np.pi)
    s1 = jnp.sin(t)
    c1 = jnp.cos(t)
    two_c1 = c1 + c1
    S = [s1, two_c1 * s1]
    C = [c1, two_c1 * c1 - 1.0]
    for _ in range(14):
        S.append(two_c1 * S[-1] - S[-2])
        C.append(two_c1 * C[-1] - C[-2])
    a = jnp.concatenate(S + C, axis=0)
    xpe = lax.dot_general(
        a, w_ref[...], (((0,), (0,)), ((), ())),
        preferred_element_type=jnp.float32,
    )
    g = g_ref[...]
    x_ref[...] = (
        g[:, 0:DIM] + g[:, DIM:2 * DIM] + g[:, 2 * DIM:3 * DIM]
        + xpe + b_ref[...]
    ) * 0.5
    msk_ref[...] = (l0_ref[...] < 0).astype(jnp.int32)


def _mixer(g192, cpT, l0, w2, b, n_rows):
    r_blk = 4096
    grid = (n_rows // r_blk,)
    return pl.pallas_call(
        _mixer_body,
        grid=grid,
        in_specs=[
            pl.BlockSpec((r_blk, 3 * DIM), lambda i: (i, 0)),
            pl.BlockSpec((8, r_blk), lambda i: (0, i)),
            pl.BlockSpec((r_blk // LANE, LANE), lambda i: (i, 0)),
            pl.BlockSpec((256, DIM), lambda i: (0, 0)),
            pl.BlockSpec((1, DIM), lambda i: (0, 0)),
        ],
        out_specs=[
            pl.BlockSpec((r_blk, DIM), lambda i: (i, 0)),
            pl.BlockSpec((r_blk // LANE, LANE), lambda i: (i, 0)),
        ],
        out_shape=[
            jax.ShapeDtypeStruct((n_rows, DIM), jnp.float32),
            jax.ShapeDtypeStruct((n_rows // LANE, LANE), jnp.int32),
        ],
    )(g192, cpT, l0, w2, b.reshape(1, DIM))


def kernel(labels, control_points, stroke_table, startpoint_table, endpoint_table, W, b):
    b_, s_ = labels.shape[0], labels.shape[1]
    n = b_ * s_
    v = stroke_table.shape[0]

    tcat = jnp.concatenate([stroke_table, startpoint_table, endpoint_table], axis=0)
    idx2d = labels.reshape(-1, LANE)
    off_np = np.fromfunction(
        lambda p, l: ((p + l) % 3) * v, (3, LANE), dtype=np.int64
    ).astype(np.int32)
    g = _gather3(tcat, idx2d, jnp.asarray(off_np), n)

    w2 = W.reshape(8, 32, DIM).transpose(1, 0, 2).reshape(16 * 16, DIM)

    cpT = control_points.reshape(n, 8).T
    l0 = labels.reshape(n, 3)[:, 0].reshape(n // LANE, LANE)
    x, msk = _mixer(g.reshape(n, 3 * DIM), cpT, l0, w2, b, n)
    return x.reshape(b_, s_, DIM), msk.reshape(b_, s_).astype(jnp.bool_)


# speedup vs baseline: 3.0274x; 1.0097x over previous
import functools

import numpy as np
import jax
import jax.numpy as jnp
from jax import lax
from jax.experimental import pallas as pl
from jax.experimental.pallas import tpu as pltpu
from jax.experimental.pallas import tpu_sc as plsc

DIM = 64
NW = 32
LANE = 128
ROWS_PC = 6
RPC = ROWS_PC * LANE


def _gather3(tcat, idx2d, off, n_tok):
    rows_pw = idx2d.shape[0] // NW
    cpw = rows_pw // ROWS_PC
    mesh = plsc.VectorSubcoreMesh(core_axis_name="c", subcore_axis_name="s")

    @functools.partial(
        pl.kernel,
        mesh=mesh,
        compiler_params=pltpu.CompilerParams(use_tc_tiling_on_sc=False),
        out_type=jax.ShapeDtypeStruct((n_tok, DIM), jnp.float32),
        scratch_types=[
            pltpu.VMEM((rows_pw, LANE), jnp.int32),
            pltpu.VMEM((3, LANE), jnp.int32),
            pltpu.VMEM((RPC, DIM), jnp.float32),
            pltpu.VMEM((RPC // 3, DIM), jnp.float32),
            pltpu.SemaphoreType.DMA,
        ],
    )
    def body(t_h, idx_h, off_h, out_h, idx_v, off_v, g_v, s_v, sem):
        wid = lax.axis_index("s") * 2 + lax.axis_index("c")
        pltpu.sync_copy(idx_h.at[pl.ds(wid * rows_pw, rows_pw)], idx_v)
        pltpu.sync_copy(off_h, off_v)

        for r3 in range(3):
            phase = (2 * r3) % 3

            def addbody(i, c, r3=r3, phase=phase):
                row = i * 3 + r3
                for s in range(LANE // 16):
                    sl = pl.ds(s * 16, 16)
                    idx_v[row, sl] = idx_v[row, sl] + off_v[phase, sl]
                return c

            lax.fori_loop(0, rows_pw // 3, addbody, 0)

        def chunk(ch, c):
            cps = []
            for r6 in range(ROWS_PC):
                cps.append(
                    pltpu.async_copy(
                        t_h.at[idx_v.at[ch * ROWS_PC + r6]],
                        g_v.at[pl.ds(r6 * LANE, LANE)],
                        sem,
                    )
                )
            for cp_ in cps:
                cp_.wait()

            def tok(k, c2):
                for s in range(DIM // 16):
                    sl = pl.ds(s * 16, 16)
                    s_v[k, sl] = (
                        g_v[3 * k, sl] + g_v[3 * k + 1, sl] + g_v[3 * k + 2, sl]
                    )
                return c2

            tpc = RPC // 3
            lax.fori_loop(0, tpc, tok, 0)
            pltpu.sync_copy(s_v, out_h.at[pl.ds((wid * cpw + ch) * tpc, tpc)])
            return c

        lax.fori_loop(0, cpw, chunk, 0)

    return body(tcat, idx2d, off)


def _mixer_body(g_ref, cpt_ref, l0_ref, w_ref, b_ref, x_ref, msk_ref):
    t = cpt_ref[...] * np.float32(np.pi)
    s1 = jnp.sin(t)
    c1 = jnp.cos(t)
    two_c1 = c1 + c1
    S = [s1, two_c1 * s1]
    C = [c1, two_c1 * c1 - 1.0]
    for _ in range(14):
        S.append(two_c1 * S[-1] - S[-2])
        C.append(two_c1 * C[-1] - C[-2])
    a = jnp.concatenate(S + C, axis=0)
    xpe = lax.dot_general(
        a, w_ref[...], (((0,), (0,)), ((), ())),
        preferred_element_type=jnp.float32,
    )
    x_ref[...] = (g_ref[...] + xpe + b_ref[...]) * 0.5
    msk_ref[...] = (l0_ref[...] < 0).astype(jnp.int32)


def _mixer(g192, cpT, l0, w2, b, n_rows):
    r_blk = 4096
    grid = (n_rows // r_blk,)
    return pl.pallas_call(
        _mixer_body,
        grid=grid,
        in_specs=[
            pl.BlockSpec((r_blk, DIM), lambda i: (i, 0)),
            pl.BlockSpec((8, r_blk), lambda i: (0, i)),
            pl.BlockSpec((r_blk // LANE, LANE), lambda i: (i, 0)),
            pl.BlockSpec((256, DIM), lambda i: (0, 0)),
            pl.BlockSpec((1, DIM), lambda i: (0, 0)),
        ],
        out_specs=[
            pl.BlockSpec((r_blk, DIM), lambda i: (i, 0)),
            pl.BlockSpec((r_blk // LANE, LANE), lambda i: (i, 0)),
        ],
        out_shape=[
            jax.ShapeDtypeStruct((n_rows, DIM), jnp.float32),
            jax.ShapeDtypeStruct((n_rows // LANE, LANE), jnp.int32),
        ],
    )(g192, cpT, l0, w2, b.reshape(1, DIM))


def kernel(labels, control_points, stroke_table, startpoint_table, endpoint_table, W, b):
    b_, s_ = labels.shape[0], labels.shape[1]
    n = b_ * s_
    v = stroke_table.shape[0]

    tcat = jnp.concatenate([stroke_table, startpoint_table, endpoint_table], axis=0)
    idx2d = labels.reshape(-1, LANE)
    off_np = np.fromfunction(
        lambda p, l: ((p + l) % 3) * v, (3, LANE), dtype=np.int64
    ).astype(np.int32)
    g = _gather3(tcat, idx2d, jnp.asarray(off_np), n)

    w2 = W.reshape(8, 32, DIM).transpose(1, 0, 2).reshape(16 * 16, DIM)

    cpT = control_points.reshape(n, 8).T
    l0 = labels.reshape(n, 3)[:, 0].reshape(n // LANE, LANE)
    x, msk = _mixer(g, cpT, l0, w2, b, n)
    return x.reshape(b_, s_, DIM), msk.reshape(b_, s_).astype(jnp.bool_)


# speedup vs baseline: 5.2311x; 1.7280x over previous
import functools

import numpy as np
import jax
import jax.numpy as jnp
from jax import lax
from jax.experimental import pallas as pl
from jax.experimental.pallas import tpu as pltpu
from jax.experimental.pallas import tpu_sc as plsc

DIM = 64
NW = 32
LANE = 128
TPC = 256


def _gather3(t0, t1, t2, idxT, n_tok):
    tpw = n_tok // NW
    cpw = tpw // TPC
    mesh = plsc.VectorSubcoreMesh(core_axis_name="c", subcore_axis_name="s")

    @functools.partial(
        pl.kernel,
        mesh=mesh,
        compiler_params=pltpu.CompilerParams(use_tc_tiling_on_sc=False),
        out_type=jax.ShapeDtypeStruct((n_tok, DIM), jnp.float32),
        scratch_types=[
            pltpu.VMEM((3, tpw), jnp.int32),
            pltpu.VMEM((3, TPC, DIM), jnp.float32),
            pltpu.VMEM((TPC, DIM), jnp.float32),
            pltpu.SemaphoreType.DMA,
        ],
    )
    def body(t0_h, t1_h, t2_h, idx_h, out_h, idx_v, g_v, s_v, sem):
        wid = lax.axis_index("s") * 2 + lax.axis_index("c")
        for t in range(3):
            pltpu.sync_copy(idx_h.at[t, pl.ds(wid * tpw, tpw)], idx_v.at[t])

        def chunk(ch, c):
            cps = []
            for t, tbl in enumerate((t0_h, t1_h, t2_h)):
                for j in range(TPC // LANE):
                    cps.append(
                        pltpu.async_copy(
                            tbl.at[idx_v.at[t, pl.ds(ch * TPC + j * LANE, LANE)]],
                            g_v.at[t, pl.ds(j * LANE, LANE)],
                            sem,
                        )
                    )
            for cp_ in cps:
                cp_.wait()

            def tok(k, c2):
                for s in range(DIM // 16):
                    sl = pl.ds(s * 16, 16)
                    s_v[k, sl] = g_v[0, k, sl] + g_v[1, k, sl] + g_v[2, k, sl]
                return c2

            lax.fori_loop(0, TPC, tok, 0)
            pltpu.sync_copy(s_v, out_h.at[pl.ds((wid * cpw + ch) * TPC, TPC)])
            return c

        lax.fori_loop(0, cpw, chunk, 0)

    return body(t0, t1, t2, idxT)


SEQ_BLK = 8


def _mixer_body(g_ref, cpt_ref, l0_ref, w_ref, b_ref, x_ref, msk_ref):
    t = cpt_ref[...] * np.float32(np.pi)
    s1 = jnp.sin(t)
    c1 = jnp.cos(t)
    two_c1 = c1 + c1
    S = [s1, two_c1 * s1]
    C = [c1, two_c1 * c1 - 1.0]
    for _ in range(14):
        S.append(two_c1 * S[-1] - S[-2])
        C.append(two_c1 * C[-1] - C[-2])
    n_bat = cpt_ref.shape[1]
    w = w_ref[...]
    bb = b_ref[...]
    for s8 in range(SEQ_BLK):
        a = jnp.concatenate(
            [v[8 * s8:8 * s8 + 8] for v in S + C], axis=0
        )
        xpe = lax.dot_general(
            a, w, (((0,), (0,)), ((), ())),
            preferred_element_type=jnp.float32,
        )
        r = pl.ds(s8 * n_bat, n_bat)
        x_ref[r, :] = (g_ref[r, :] + xpe + bb) * 0.5
    msk_ref[...] = (l0_ref[...] < 0).astype(jnp.int32)


def _mixer(g, cpX, l0, w2, b, n_seq, n_bat):
    grid = (n_seq // SEQ_BLK,)
    return pl.pallas_call(
        _mixer_body,
        grid=grid,
        in_specs=[
            pl.BlockSpec((SEQ_BLK * n_bat, DIM), lambda i: (i, 0)),
            pl.BlockSpec((8 * SEQ_BLK, n_bat), lambda i: (i, 0)),
            pl.BlockSpec((SEQ_BLK, n_bat), lambda i: (i, 0)),
            pl.BlockSpec((256, DIM), lambda i: (0, 0)),
            pl.BlockSpec((1, DIM), lambda i: (0, 0)),
        ],
        out_specs=[
            pl.BlockSpec((SEQ_BLK * n_bat, DIM), lambda i: (i, 0)),
            pl.BlockSpec((SEQ_BLK, n_bat), lambda i: (i, 0)),
        ],
        out_shape=[
            jax.ShapeDtypeStruct((n_seq * n_bat, DIM), jnp.float32),
            jax.ShapeDtypeStruct((n_seq, n_bat), jnp.int32),
        ],
    )(g, cpX, l0, w2, b.reshape(1, DIM))


def kernel(labels, control_points, stroke_table, startpoint_table, endpoint_table, W, b):
    b_, s_ = labels.shape[0], labels.shape[1]
    n = b_ * s_

    idxT = jnp.transpose(labels, (2, 1, 0)).reshape(3, n)
    cpX = jnp.transpose(control_points, (1, 2, 3, 0)).reshape(s_ * 8, b_)
    l0 = jnp.transpose(labels[:, :, 0], (1, 0))

    g = _gather3(stroke_table, startpoint_table, endpoint_table, idxT, n)

    w2 = W.reshape(8, 32, DIM).transpose(1, 0, 2).reshape(256, DIM)

    x, msk = _mixer(g, cpX, l0, w2, b, s_, b_)
    x = jnp.transpose(x.reshape(s_, b_, DIM), (1, 0, 2))
    return x, jnp.transpose(msk, (1, 0)).astype(jnp.bool_)
